# Initial kernel scaffold; baseline (speedup 1.0000x reference)
import jax
import jax.numpy as jnp
from jax.experimental import pallas as pl


def kernel(H, edge_index, edge_values, W1, b1, Wl, bl):
    raise NotImplementedError("write your pallas kernel here")


# speedup vs baseline: 2.7406x; 2.7406x over previous
import functools

import jax
import jax.numpy as jnp
from jax import lax
from jax.experimental import pallas as pl
from jax.experimental.pallas import tpu as pltpu
from jax.experimental.pallas import tpu_sc as plsc

N = 10000
E = 320000
D_IN = 128
D_HID = 64
G = 16

NC = 2
NS = 16
NW = NC * NS
EPT = E // NW
C = 80
NCHUNK = EPT // C
NPAD = 10240
RPW = NPAD // NS

_f32 = jnp.float32
_i32 = jnp.int32


def _t1_body(h_ref, w_ref, o_ref):
    o_ref[...] = jnp.dot(h_ref[...], w_ref[...], preferred_element_type=_f32)


_t1 = pl.pallas_call(
    _t1_body,
    out_shape=jax.ShapeDtypeStruct((N, D_HID), _f32),
)


@functools.lru_cache(maxsize=None)
def _build_s1():
  mesh = plsc.VectorSubcoreMesh(
      core_axis_name="c", subcore_axis_name="s", num_cores=NC, num_subcores=NS
  )

  @functools.partial(
    pl.kernel,
    out_type=(
        jax.ShapeDtypeStruct((NC, NPAD, D_HID), _f32),
        jax.ShapeDtypeStruct((NC, NPAD, 16), _f32),
    ),
    mesh=mesh,
    compiler_params=pltpu.CompilerParams(needs_layout_passes=False, use_tc_tiling_on_sc=False),
    scratch_types=[
        pltpu.VMEM((C,), _i32),
        pltpu.VMEM((C,), _i32),
        pltpu.VMEM((C,), _f32),
        pltpu.VMEM((C, D_HID), _f32),
        pltpu.VMEM((C, 16), _f32),
        pltpu.VMEM_SHARED((NPAD, D_HID), _f32),
        pltpu.VMEM_SHARED((NPAD, 16), _f32),
        pltpu.SemaphoreType.DMA,
    ],
  )
  def _s1(hw, row, col, val, zrows, zd, out_ahw, out_d,
          rowv, colv, valv, rows, valb, acc, accd, sem):
    c = lax.axis_index("c")
    s = lax.axis_index("s")
    wid = c * NS + s
    rslice = pl.ds(s * RPW, RPW)

    pltpu.sync_copy(zrows.at[rslice], acc.at[rslice])
    pltpu.sync_copy(zd.at[rslice], accd.at[rslice])
    pltpu.sync_copy(zd.at[pl.ds(0, C)], valb)
    plsc.subcore_barrier()

    def chunk(i, carry):
        base = wid * EPT + i * C
        pltpu.sync_copy(row.at[pl.ds(base, C)], rowv)
        pltpu.sync_copy(col.at[pl.ds(base, C)], colv)
        pltpu.sync_copy(val.at[pl.ds(base, C)], valv)
        pltpu.async_copy(hw.at[colv], rows, sem).wait()
        zeros16 = jnp.zeros((16,), _i32)
        for g in range(C // 16):
            vals16 = valv[pl.ds(g * 16, 16)]
            eidx = lax.iota(_i32, 16) + g * 16
            plsc.store_scatter(valb, [eidx, zeros16], vals16)
            for j in range(D_HID):
                jv = jnp.full((16,), j, _i32)
                x = plsc.load_gather(rows, [eidx, jv])
                plsc.store_scatter(rows, [eidx, jv], x * vals16)
        pltpu.sync_copy(rows, acc.at[rowv], add=True)
        pltpu.sync_copy(valb, accd.at[rowv], add=True)
        return carry

    lax.fori_loop(0, NCHUNK, chunk, 0)
    plsc.subcore_barrier()
    pltpu.sync_copy(acc.at[rslice], out_ahw.at[c].at[rslice])
    pltpu.sync_copy(accd.at[rslice], out_d.at[c].at[rslice])

  return _s1


def _t2_body(ahw_ref, d_ref, b1_ref, wlt_ref, bl_ref, ybg_ref, y1m_ref):
    ahw = ahw_ref[0, :N, :] + ahw_ref[1, :N, :]
    h1 = jnp.maximum(ahw + b1_ref[...], 0.0)
    h2 = jnp.dot(h1, wlt_ref[...], preferred_element_type=_f32) + bl_ref[...]
    h2 = jnp.maximum(h2, 0.0)
    m = jnp.max(h2, axis=1, keepdims=True)
    ex = jnp.exp(h2 - m)
    y = ex / jnp.sum(ex, axis=1, keepdims=True)
    d = d_ref[0, :N, 0] + d_ref[1, :N, 0]
    gamma = jnp.sum(y * d[:, None], axis=0, keepdims=True)
    ybg_ref[...] = y / gamma
    y1m_ref[...] = 1.0 - y


_t2 = pl.pallas_call(
    _t2_body,
    out_shape=(
        jax.ShapeDtypeStruct((N, G), _f32),
        jax.ShapeDtypeStruct((N, G), _f32),
    ),
)


@functools.lru_cache(maxsize=None)
def _build_s2():
  mesh = plsc.VectorSubcoreMesh(
      core_axis_name="c", subcore_axis_name="s", num_cores=NC, num_subcores=NS
  )

  @functools.partial(
    pl.kernel,
    out_type=jax.ShapeDtypeStruct((NW, 16), _f32),
    mesh=mesh,
    compiler_params=pltpu.CompilerParams(needs_layout_passes=False, use_tc_tiling_on_sc=False),
    scratch_types=[
        pltpu.VMEM((C,), _i32),
        pltpu.VMEM((C,), _i32),
        pltpu.VMEM((C,), _f32),
        pltpu.VMEM((C, 16), _f32),
        pltpu.VMEM((C, 16), _f32),
        pltpu.VMEM((16,), _f32),
        pltpu.SemaphoreType.DMA,
        pltpu.SemaphoreType.DMA,
    ],
  )
  def _s2(ybg, y1m, row, col, val, out,
          rowv, colv, valv, ybgr, y1mr, accb, sem1, sem2):
    c = lax.axis_index("c")
    s = lax.axis_index("s")
    wid = c * NS + s

    def chunk(i, acc):
        base = wid * EPT + i * C
        pltpu.sync_copy(row.at[pl.ds(base, C)], rowv)
        pltpu.sync_copy(col.at[pl.ds(base, C)], colv)
        pltpu.sync_copy(val.at[pl.ds(base, C)], valv)
        d1 = pltpu.async_copy(ybg.at[rowv], ybgr, sem1)
        d2 = pltpu.async_copy(y1m.at[colv], y1mr, sem2)
        d1.wait()
        d2.wait()
        for e in range(C):
            vs = plsc.load_gather(valv, [jnp.full((16,), e, _i32)])
            acc = acc + vs * ybgr[e] * y1mr[e]
        return acc

    acc = lax.fori_loop(0, NCHUNK, chunk, jnp.zeros((16,), _f32))
    accb[...] = acc
    pltpu.sync_copy(accb, out.at[wid])

  return _s2


def kernel(H, edge_index, edge_values, W1, b1, Wl, bl):
    row = edge_index[0]
    col = edge_index[1]
    hw = _t1(H, W1)
    zrows = jnp.zeros((NPAD, D_HID), _f32)
    zd = jnp.zeros((NPAD, 16), _f32)
    ahwp, dp = _build_s1()(hw, row, col, edge_values, zrows, zd)
    ybg, y1m = _t2(ahwp, dp, b1.reshape(1, D_HID), Wl.T, bl.reshape(1, G))
    parts = _build_s2()(ybg, y1m, row, col, edge_values)
    return jnp.sum(parts).reshape(1)


# speedup vs baseline: 3.9762x; 1.4508x over previous
import functools

import jax
import jax.numpy as jnp
from jax import lax
from jax.experimental import pallas as pl
from jax.experimental.pallas import tpu as pltpu
from jax.experimental.pallas import tpu_sc as plsc

N = 10000
E = 320000
D_IN = 128
D_HID = 64
G = 16

NC = 2
NS = 16
NW = NC * NS
EPT = E // NW
C = 80
NCHUNK = EPT // C
NPAD = 10240
RPW = NPAD // NS
K = 5

_f32 = jnp.float32
_i32 = jnp.int32

_sc_params = pltpu.CompilerParams(
    needs_layout_passes=False, use_tc_tiling_on_sc=False
)


def _t1_body(h_ref, w_ref, o_ref):
    o_ref[...] = jnp.dot(h_ref[...], w_ref[...], preferred_element_type=_f32)


_t1 = pl.pallas_call(
    _t1_body,
    out_shape=jax.ShapeDtypeStruct((N, D_HID), _f32),
)


@functools.lru_cache(maxsize=None)
def _build_s1():
  mesh = plsc.VectorSubcoreMesh(
      core_axis_name="c", subcore_axis_name="s", num_cores=NC, num_subcores=NS
  )

  @functools.partial(
    pl.kernel,
    out_type=(
        jax.ShapeDtypeStruct((NC, NPAD, D_HID), _f32),
        jax.ShapeDtypeStruct((NW, NPAD), _f32),
    ),
    mesh=mesh,
    compiler_params=_sc_params,
    scratch_types=[
        pltpu.VMEM((NCHUNK, C), _i32),
        pltpu.VMEM((NCHUNK, C), _i32),
        pltpu.VMEM((NCHUNK, C), _f32),
        [pltpu.VMEM((C, D_HID), _f32) for _ in range(K)],
        pltpu.VMEM((NPAD,), _f32),
        pltpu.VMEM_SHARED((NPAD, D_HID), _f32),
        [pltpu.SemaphoreType.DMA for _ in range(K)],
        [pltpu.SemaphoreType.DMA for _ in range(K)],
    ],
  )
  def _s1(hw, row3, col3, val3, zrows, zd1, out_ahw, out_d,
          rowa, cola, vala, rows, dloc, acc, gsem, ssem):
    c = lax.axis_index("c")
    s = lax.axis_index("s")
    wid = c * NS + s
    rslice = pl.ds(s * RPW, RPW)

    pltpu.sync_copy(row3.at[wid], rowa)
    pltpu.sync_copy(col3.at[wid], cola)
    pltpu.sync_copy(val3.at[wid], vala)
    pltpu.sync_copy(zrows.at[rslice], acc.at[rslice])
    pltpu.sync_copy(zd1, dloc)
    plsc.subcore_barrier()

    def issue_gather(j, b):
        pltpu.async_copy(hw.at[cola.at[j]], rows[b], gsem[b])

    def drain_gather(b):
        pltpu.make_async_copy(hw.at[pl.ds(0, C)], rows[b], gsem[b]).wait()

    def drain_scatter(b):
        pltpu.make_async_copy(hw.at[pl.ds(0, C)], rows[b], ssem[b]).wait()

    for b in range(K - 1):
        issue_gather(b, b)

    def step(tt, b):
        i = tt * K + b
        j = i + (K - 1)
        bj = (b + K - 1) % K

        def fire():
            if b == 0:
                @pl.when(tt >= 1)
                def _():
                    drain_scatter(bj)
            else:
                drain_scatter(bj)
            issue_gather(j, bj)

        if b == 0:
            fire()
        else:
            @pl.when(j < NCHUNK)
            def _():
                fire()

        drain_gather(b)
        for g in range(C // 16):
            vals16 = vala[i, pl.ds(g * 16, 16)]
            row16 = rowa[i, pl.ds(g * 16, 16)]
            plsc.addupdate_scatter(dloc, [row16], vals16)
            eidx = lax.iota(_i32, 16) + g * 16
            for jf in range(D_HID):
                jv = jnp.full((16,), jf, _i32)
                x = plsc.load_gather(rows[b], [eidx, jv])
                plsc.store_scatter(rows[b], [eidx, jv], x * vals16)
        pltpu.async_copy(rows[b], acc.at[rowa.at[i]], ssem[b], add=True)

    def body(tt, carry):
        for b in range(K):
            step(tt, b)
        return carry

    lax.fori_loop(0, NCHUNK // K, body, 0)
    for b in range(K):
        drain_scatter(b)
    plsc.subcore_barrier()
    pltpu.sync_copy(acc.at[rslice], out_ahw.at[c].at[rslice])
    pltpu.sync_copy(dloc, out_d.at[wid])

  return _s1


def _t2_body(ahw_ref, d_ref, b1_ref, wlt_ref, bl_ref, ybg_ref, y1m_ref):
    ahw = ahw_ref[0, :N, :] + ahw_ref[1, :N, :]
    h1 = jnp.maximum(ahw + b1_ref[...], 0.0)
    h2 = jnp.dot(h1, wlt_ref[...], preferred_element_type=_f32) + bl_ref[...]
    h2 = jnp.maximum(h2, 0.0)
    m = jnp.max(h2, axis=1, keepdims=True)
    ex = jnp.exp(h2 - m)
    y = ex / jnp.sum(ex, axis=1, keepdims=True)
    d = jnp.sum(d_ref[:, :N], axis=0)
    gamma = jnp.sum(y * d[:, None], axis=0, keepdims=True)
    ybg_ref[...] = y / gamma
    y1m_ref[...] = 1.0 - y


_t2 = pl.pallas_call(
    _t2_body,
    out_shape=(
        jax.ShapeDtypeStruct((N, G), _f32),
        jax.ShapeDtypeStruct((N, G), _f32),
    ),
)


@functools.lru_cache(maxsize=None)
def _build_s2():
  mesh = plsc.VectorSubcoreMesh(
      core_axis_name="c", subcore_axis_name="s", num_cores=NC, num_subcores=NS
  )

  @functools.partial(
    pl.kernel,
    out_type=jax.ShapeDtypeStruct((NW, 16), _f32),
    mesh=mesh,
    compiler_params=_sc_params,
    scratch_types=[
        pltpu.VMEM((NCHUNK, C), _i32),
        pltpu.VMEM((NCHUNK, C), _i32),
        pltpu.VMEM((NCHUNK, C), _f32),
        [pltpu.VMEM((C, G), _f32) for _ in range(K)],
        [pltpu.VMEM((C, G), _f32) for _ in range(K)],
        pltpu.VMEM((16,), _f32),
        [pltpu.SemaphoreType.DMA for _ in range(K)],
        [pltpu.SemaphoreType.DMA for _ in range(K)],
    ],
  )
  def _s2(ybg, y1m, row3, col3, val3, out,
          rowa, cola, vala, ybgr, y1mr, accb, ysem, zsem):
    c = lax.axis_index("c")
    s = lax.axis_index("s")
    wid = c * NS + s

    pltpu.sync_copy(row3.at[wid], rowa)
    pltpu.sync_copy(col3.at[wid], cola)
    pltpu.sync_copy(val3.at[wid], vala)

    def issue_gathers(j, b):
        pltpu.async_copy(ybg.at[rowa.at[j]], ybgr[b], ysem[b])
        pltpu.async_copy(y1m.at[cola.at[j]], y1mr[b], zsem[b])

    def drain_gathers(b):
        pltpu.make_async_copy(ybg.at[pl.ds(0, C)], ybgr[b], ysem[b]).wait()
        pltpu.make_async_copy(y1m.at[pl.ds(0, C)], y1mr[b], zsem[b]).wait()

    for b in range(K - 1):
        issue_gathers(b, b)

    def step(tt, b, acc):
        i = tt * K + b
        j = i + (K - 1)
        bj = (b + K - 1) % K
        if b == 0:
            issue_gathers(j, bj)
        else:
            @pl.when(j < NCHUNK)
            def _():
                issue_gathers(j, bj)
        drain_gathers(b)
        for e in range(C):
            vs = plsc.load_gather(
                vala, [jnp.full((16,), i, _i32), jnp.full((16,), e, _i32)]
            )
            acc = acc + vs * ybgr[b][e] * y1mr[b][e]
        return acc

    def body(tt, acc):
        for b in range(K):
            acc = step(tt, b, acc)
        return acc

    acc = lax.fori_loop(0, NCHUNK // K, body, jnp.zeros((16,), _f32))
    accb[...] = acc
    pltpu.sync_copy(accb, out.at[wid])

  return _s2


def kernel(H, edge_index, edge_values, W1, b1, Wl, bl):
    row3 = edge_index[0].reshape(NW, NCHUNK, C)
    col3 = edge_index[1].reshape(NW, NCHUNK, C)
    val3 = edge_values.reshape(NW, NCHUNK, C)
    hw = _t1(H, W1)
    zrows = jnp.zeros((NPAD, D_HID), _f32)
    zd1 = jnp.zeros((NPAD,), _f32)
    ahwp, dp = _build_s1()(hw, row3, col3, val3, zrows, zd1)
    ybg, y1m = _t2(ahwp, dp, b1.reshape(1, D_HID), Wl.T, bl.reshape(1, G))
    parts = _build_s2()(ybg, y1m, row3, col3, val3)
    return jnp.sum(parts).reshape(1)


# speedup vs baseline: 4.1693x; 1.0486x over previous
import functools

import jax
import jax.numpy as jnp
from jax import lax
from jax.experimental import pallas as pl
from jax.experimental.pallas import tpu as pltpu
from jax.experimental.pallas import tpu_sc as plsc

N = 10000
E = 320000
D_IN = 128
D_HID = 64
G = 16

NC = 2
NS = 16
NW = NC * NS
EPT = E // NW
C = 80
NCHUNK = EPT // C
NPAD = 10240
RPW = NPAD // NS
K = 5

_f32 = jnp.float32
_i32 = jnp.int32

_sc_params = pltpu.CompilerParams(
    needs_layout_passes=False, use_tc_tiling_on_sc=False
)


def _t1_body(h_ref, w_ref, o_ref):
    o_ref[...] = jnp.dot(h_ref[...], w_ref[...], preferred_element_type=_f32)


_t1 = pl.pallas_call(
    _t1_body,
    out_shape=jax.ShapeDtypeStruct((N, D_HID), _f32),
)


@functools.lru_cache(maxsize=None)
def _build_s1():
  mesh = plsc.VectorSubcoreMesh(
      core_axis_name="c", subcore_axis_name="s", num_cores=NC, num_subcores=NS
  )

  @functools.partial(
    pl.kernel,
    out_type=jax.ShapeDtypeStruct((NC, NPAD, D_HID), _f32),
    mesh=mesh,
    compiler_params=_sc_params,
    scratch_types=[
        pltpu.VMEM((NCHUNK, C), _i32),
        pltpu.VMEM((NCHUNK, C), _i32),
        pltpu.VMEM((NCHUNK, C), _f32),
        [pltpu.VMEM((C, D_HID), _f32) for _ in range(K)],
        [pltpu.VMEM((C, D_HID), _f32) for _ in range(K)],
        pltpu.VMEM_SHARED((NPAD, D_HID), _f32),
        [pltpu.SemaphoreType.DMA for _ in range(K)],
        [pltpu.SemaphoreType.DMA for _ in range(K)],
    ],
  )
  def _s1(hw, row3, col3, val3, zrows, out_ahw,
          rowa, cola, vala, rows, rowso, acc, gsem, ssem):
    c = lax.axis_index("c")
    s = lax.axis_index("s")
    wid = c * NS + s
    rslice = pl.ds(s * RPW, RPW)

    pltpu.sync_copy(row3.at[wid], rowa)
    pltpu.sync_copy(col3.at[wid], cola)
    pltpu.sync_copy(val3.at[wid], vala)
    pltpu.sync_copy(zrows.at[rslice], acc.at[rslice])
    plsc.subcore_barrier()

    def issue_gather(j, b):
        pltpu.async_copy(hw.at[cola.at[j]], rows[b], gsem[b])

    def drain_gather(b):
        pltpu.make_async_copy(hw.at[pl.ds(0, C)], rows[b], gsem[b]).wait()

    def drain_scatter(b):
        pltpu.make_async_copy(hw.at[pl.ds(0, C)], rowso[b], ssem[b]).wait()

    for b in range(K - 1):
        issue_gather(b, b)

    def step(tt, b):
        i = tt * K + b
        j = i + (K - 1)
        bj = (b + K - 1) % K

        if b == 0:
            issue_gather(j, bj)
        else:
            @pl.when(j < NCHUNK)
            def _():
                issue_gather(j, bj)

        drain_gather(b)
        @pl.when(tt >= 1)
        def _():
            drain_scatter(b)
        for g in range(C // 16):
            vals16 = vala[i, pl.ds(g * 16, 16)]
            eidx = lax.iota(_i32, 16) + g * 16
            for jf in range(D_HID):
                jv = jnp.full((16,), jf, _i32)
                x = plsc.load_gather(rows[b], [eidx, jv])
                plsc.store_scatter(rowso[b], [eidx, jv], x * vals16)
        pltpu.async_copy(rowso[b], acc.at[rowa.at[i]], ssem[b], add=True)

    def body(tt, carry):
        for b in range(K):
            step(tt, b)
        return carry

    lax.fori_loop(0, NCHUNK // K, body, 0)
    for b in range(K):
        drain_scatter(b)
    plsc.subcore_barrier()
    pltpu.sync_copy(acc.at[rslice], out_ahw.at[c].at[rslice])

  return _s1


def _t2_body(ahw_ref, b1_ref, wlt_ref, bl_ref, y_ref, y1m_ref):
    ahw = ahw_ref[0, :N, :] + ahw_ref[1, :N, :]
    h1 = jnp.maximum(ahw + b1_ref[...], 0.0)
    h2 = jnp.dot(h1, wlt_ref[...], preferred_element_type=_f32) + bl_ref[...]
    h2 = jnp.maximum(h2, 0.0)
    m = jnp.max(h2, axis=1, keepdims=True)
    ex = jnp.exp(h2 - m)
    y = ex / jnp.sum(ex, axis=1, keepdims=True)
    y_ref[...] = y
    y1m_ref[...] = 1.0 - y


_t2 = pl.pallas_call(
    _t2_body,
    out_shape=(
        jax.ShapeDtypeStruct((N, G), _f32),
        jax.ShapeDtypeStruct((N, G), _f32),
    ),
)


@functools.lru_cache(maxsize=None)
def _build_s2():
  mesh = plsc.VectorSubcoreMesh(
      core_axis_name="c", subcore_axis_name="s", num_cores=NC, num_subcores=NS
  )

  @functools.partial(
    pl.kernel,
    out_type=jax.ShapeDtypeStruct((NW, 2, 16), _f32),
    mesh=mesh,
    compiler_params=_sc_params,
    scratch_types=[
        pltpu.VMEM((NCHUNK, C), _i32),
        pltpu.VMEM((NCHUNK, C), _i32),
        pltpu.VMEM((NCHUNK, C), _f32),
        [pltpu.VMEM((C, G), _f32) for _ in range(K)],
        [pltpu.VMEM((C, G), _f32) for _ in range(K)],
        pltpu.VMEM((2, 16), _f32),
        [pltpu.SemaphoreType.DMA for _ in range(K)],
        [pltpu.SemaphoreType.DMA for _ in range(K)],
    ],
  )
  def _s2(ybg, y1m, row3, col3, val3, out,
          rowa, cola, vala, ybgr, y1mr, accb, ysem, zsem):
    c = lax.axis_index("c")
    s = lax.axis_index("s")
    wid = c * NS + s

    pltpu.sync_copy(row3.at[wid], rowa)
    pltpu.sync_copy(col3.at[wid], cola)
    pltpu.sync_copy(val3.at[wid], vala)

    def issue_gathers(j, b):
        pltpu.async_copy(ybg.at[rowa.at[j]], ybgr[b], ysem[b])
        pltpu.async_copy(y1m.at[cola.at[j]], y1mr[b], zsem[b])

    def drain_gathers(b):
        pltpu.make_async_copy(ybg.at[pl.ds(0, C)], ybgr[b], ysem[b]).wait()
        pltpu.make_async_copy(y1m.at[pl.ds(0, C)], y1mr[b], zsem[b]).wait()

    for b in range(K - 1):
        issue_gathers(b, b)

    def step(tt, b, carry):
        macc, gacc = carry
        i = tt * K + b
        j = i + (K - 1)
        bj = (b + K - 1) % K
        if b == 0:
            issue_gathers(j, bj)
        else:
            @pl.when(j < NCHUNK)
            def _():
                issue_gathers(j, bj)
        drain_gathers(b)
        for e in range(C):
            vs = plsc.load_gather(
                vala, [jnp.full((16,), i, _i32), jnp.full((16,), e, _i32)]
            )
            t = vs * ybgr[b][e]
            gacc = gacc + t
            macc = macc + t * y1mr[b][e]
        return macc, gacc

    def body(tt, carry):
        for b in range(K):
            carry = step(tt, b, carry)
        return carry

    macc, gacc = lax.fori_loop(
        0, NCHUNK // K, body,
        (jnp.zeros((16,), _f32), jnp.zeros((16,), _f32)),
    )
    accb[0] = macc
    accb[1] = gacc
    pltpu.sync_copy(accb, out.at[wid])

  return _s2


def kernel(H, edge_index, edge_values, W1, b1, Wl, bl):
    row3 = edge_index[0].reshape(NW, NCHUNK, C)
    col3 = edge_index[1].reshape(NW, NCHUNK, C)
    val3 = edge_values.reshape(NW, NCHUNK, C)
    hw = _t1(H, W1)
    zrows = jnp.zeros((NPAD, D_HID), _f32)
    ahwp = _build_s1()(hw, row3, col3, val3, zrows)
    y, y1m = _t2(ahwp, b1.reshape(1, D_HID), Wl.T, bl.reshape(1, G))
    parts = _build_s2()(y, y1m, row3, col3, val3)
    m = jnp.sum(parts[:, 0, :], axis=0)
    gamma = jnp.sum(parts[:, 1, :], axis=0)
    return jnp.sum(m / gamma).reshape(1)


# speedup vs baseline: 11.3681x; 2.7266x over previous
import functools

import jax
import jax.numpy as jnp
from jax import lax
from jax.experimental import pallas as pl
from jax.experimental.pallas import tpu as pltpu
from jax.experimental.pallas import tpu_sc as plsc

N = 10000
E = 320000
D_IN = 128
D_HID = 64
G = 16

NC = 2
NS = 16
NW = NC * NS
EPT = E // NW
C = 80
NCHUNK = EPT // C
NPAD = 10240
RPW = NPAD // NS
K = 5

_f32 = jnp.float32
_i32 = jnp.int32

_sc_params = pltpu.CompilerParams(
    needs_layout_passes=False, use_tc_tiling_on_sc=False
)


def _t1_body(h_ref, w_ref, o_ref):
    o_ref[...] = jnp.dot(h_ref[...], w_ref[...], preferred_element_type=_f32)


_t1 = pl.pallas_call(
    _t1_body,
    out_shape=jax.ShapeDtypeStruct((N, D_HID), _f32),
)


@functools.lru_cache(maxsize=None)
def _build_s1():
  mesh = plsc.VectorSubcoreMesh(
      core_axis_name="c", subcore_axis_name="s", num_cores=NC, num_subcores=NS
  )

  @functools.partial(
    pl.kernel,
    out_type=jax.ShapeDtypeStruct((NC, NPAD, D_HID), _f32),
    mesh=mesh,
    compiler_params=_sc_params,
    scratch_types=[
        pltpu.VMEM((NCHUNK, C), _i32),
        pltpu.VMEM((NCHUNK, C), _i32),
        pltpu.VMEM((NCHUNK, C), _f32),
        [pltpu.VMEM((C, D_HID), _f32) for _ in range(K)],
        [pltpu.VMEM((C, D_HID), _f32) for _ in range(K)],
        pltpu.VMEM_SHARED((NPAD, D_HID), _f32),
        [pltpu.SemaphoreType.DMA for _ in range(K)],
        [pltpu.SemaphoreType.DMA for _ in range(K)],
    ],
  )
  def _s1(hw, row3, col3, val3, zrows, out_ahw,
          rowa, cola, vala, rows, rowso, acc, gsem, ssem):
    c = lax.axis_index("c")
    s = lax.axis_index("s")
    wid = c * NS + s
    rslice = pl.ds(s * RPW, RPW)

    pltpu.sync_copy(row3.at[wid], rowa)
    pltpu.sync_copy(col3.at[wid], cola)
    pltpu.sync_copy(val3.at[wid], vala)
    pltpu.sync_copy(zrows.at[rslice], acc.at[rslice])
    plsc.subcore_barrier()

    def issue_gather(j, b):
        pltpu.async_copy(hw.at[cola.at[j]], rows[b], gsem[b])

    def drain_gather(b):
        pltpu.make_async_copy(hw.at[pl.ds(0, C)], rows[b], gsem[b]).wait()

    def drain_scatter(b):
        pltpu.make_async_copy(hw.at[pl.ds(0, C)], rowso[b], ssem[b]).wait()

    for b in range(K - 1):
        issue_gather(b, b)

    def step(tt, b):
        i = tt * K + b
        j = i + (K - 1)
        bj = (b + K - 1) % K

        if b == 0:
            issue_gather(j, bj)
        else:
            @pl.when(j < NCHUNK)
            def _():
                issue_gather(j, bj)

        drain_gather(b)
        @pl.when(tt >= 1)
        def _():
            drain_scatter(b)
        ivec = jnp.full((16,), i, _i32)
        for e in range(C):
            vs = plsc.load_gather(vala, [ivec, jnp.full((16,), e, _i32)])
            for kk in range(D_HID // 16):
                sl = pl.ds(kk * 16, 16)
                rowso[b][e, sl] = rows[b][e, sl] * vs
        pltpu.async_copy(rowso[b], acc.at[rowa.at[i]], ssem[b], add=True)

    def body(tt, carry):
        for b in range(K):
            step(tt, b)
        return carry

    lax.fori_loop(0, NCHUNK // K, body, 0)
    for b in range(K):
        drain_scatter(b)
    plsc.subcore_barrier()
    pltpu.sync_copy(acc.at[rslice], out_ahw.at[c].at[rslice])

  return _s1


def _t2_body(ahw_ref, b1_ref, wlt_ref, bl_ref, y_ref, y1m_ref):
    ahw = ahw_ref[0, :N, :] + ahw_ref[1, :N, :]
    h1 = jnp.maximum(ahw + b1_ref[...], 0.0)
    h2 = jnp.dot(h1, wlt_ref[...], preferred_element_type=_f32) + bl_ref[...]
    h2 = jnp.maximum(h2, 0.0)
    m = jnp.max(h2, axis=1, keepdims=True)
    ex = jnp.exp(h2 - m)
    y = ex / jnp.sum(ex, axis=1, keepdims=True)
    y_ref[...] = y
    y1m_ref[...] = 1.0 - y


_t2 = pl.pallas_call(
    _t2_body,
    out_shape=(
        jax.ShapeDtypeStruct((N, G), _f32),
        jax.ShapeDtypeStruct((N, G), _f32),
    ),
)


@functools.lru_cache(maxsize=None)
def _build_s2():
  mesh = plsc.VectorSubcoreMesh(
      core_axis_name="c", subcore_axis_name="s", num_cores=NC, num_subcores=NS
  )

  @functools.partial(
    pl.kernel,
    out_type=jax.ShapeDtypeStruct((NW, 2, 16), _f32),
    mesh=mesh,
    compiler_params=_sc_params,
    scratch_types=[
        pltpu.VMEM((NCHUNK, C), _i32),
        pltpu.VMEM((NCHUNK, C), _i32),
        pltpu.VMEM((NCHUNK, C), _f32),
        [pltpu.VMEM((C, G), _f32) for _ in range(K)],
        [pltpu.VMEM((C, G), _f32) for _ in range(K)],
        pltpu.VMEM((2, 16), _f32),
        [pltpu.SemaphoreType.DMA for _ in range(K)],
        [pltpu.SemaphoreType.DMA for _ in range(K)],
    ],
  )
  def _s2(ybg, y1m, row3, col3, val3, out,
          rowa, cola, vala, ybgr, y1mr, accb, ysem, zsem):
    c = lax.axis_index("c")
    s = lax.axis_index("s")
    wid = c * NS + s

    pltpu.sync_copy(row3.at[wid], rowa)
    pltpu.sync_copy(col3.at[wid], cola)
    pltpu.sync_copy(val3.at[wid], vala)

    def issue_gathers(j, b):
        pltpu.async_copy(ybg.at[rowa.at[j]], ybgr[b], ysem[b])
        pltpu.async_copy(y1m.at[cola.at[j]], y1mr[b], zsem[b])

    def drain_gathers(b):
        pltpu.make_async_copy(ybg.at[pl.ds(0, C)], ybgr[b], ysem[b]).wait()
        pltpu.make_async_copy(y1m.at[pl.ds(0, C)], y1mr[b], zsem[b]).wait()

    for b in range(K - 1):
        issue_gathers(b, b)

    def step(tt, b, carry):
        macc, gacc = carry
        i = tt * K + b
        j = i + (K - 1)
        bj = (b + K - 1) % K
        if b == 0:
            issue_gathers(j, bj)
        else:
            @pl.when(j < NCHUNK)
            def _():
                issue_gathers(j, bj)
        drain_gathers(b)
        for e in range(C):
            vs = plsc.load_gather(
                vala, [jnp.full((16,), i, _i32), jnp.full((16,), e, _i32)]
            )
            t = vs * ybgr[b][e]
            gacc = gacc + t
            macc = macc + t * y1mr[b][e]
        return macc, gacc

    def body(tt, carry):
        for b in range(K):
            carry = step(tt, b, carry)
        return carry

    macc, gacc = lax.fori_loop(
        0, NCHUNK // K, body,
        (jnp.zeros((16,), _f32), jnp.zeros((16,), _f32)),
    )
    accb[0] = macc
    accb[1] = gacc
    pltpu.sync_copy(accb, out.at[wid])

  return _s2


def kernel(H, edge_index, edge_values, W1, b1, Wl, bl):
    row3 = edge_index[0].reshape(NW, NCHUNK, C)
    col3 = edge_index[1].reshape(NW, NCHUNK, C)
    val3 = edge_values.reshape(NW, NCHUNK, C)
    hw = _t1(H, W1)
    zrows = jnp.zeros((NPAD, D_HID), _f32)
    ahwp = _build_s1()(hw, row3, col3, val3, zrows)
    y, y1m = _t2(ahwp, b1.reshape(1, D_HID), Wl.T, bl.reshape(1, G))
    parts = _build_s2()(y, y1m, row3, col3, val3)
    m = jnp.sum(parts[:, 0, :], axis=0)
    gamma = jnp.sum(parts[:, 1, :], axis=0)
    return jnp.sum(m / gamma).reshape(1)


# speedup vs baseline: 13.4833x; 1.1861x over previous
import functools

import jax
import jax.numpy as jnp
from jax import lax
from jax.experimental import pallas as pl
from jax.experimental.pallas import tpu as pltpu
from jax.experimental.pallas import tpu_sc as plsc

N = 10000
E = 320000
D_IN = 128
D_HID = 64
G = 16

NC = 2
NS = 16
NW = NC * NS
EPT = E // NW
C = 80
NCHUNK = EPT // C
NPAD = 10240
RPW = NPAD // NS
K = 5

_f32 = jnp.float32
_i32 = jnp.int32

_sc_params = pltpu.CompilerParams(
    needs_layout_passes=False, use_tc_tiling_on_sc=False
)


def _t1_body(h_ref, w_ref, o_ref):
    o_ref[...] = jnp.dot(h_ref[...], w_ref[...], preferred_element_type=_f32)


_t1 = pl.pallas_call(
    _t1_body,
    out_shape=jax.ShapeDtypeStruct((N, D_HID), _f32),
)


@functools.lru_cache(maxsize=None)
def _build_s1():
  mesh = plsc.VectorSubcoreMesh(
      core_axis_name="c", subcore_axis_name="s", num_cores=NC, num_subcores=NS
  )

  @functools.partial(
    pl.kernel,
    out_type=jax.ShapeDtypeStruct((NC, NPAD, D_HID), _f32),
    mesh=mesh,
    compiler_params=_sc_params,
    scratch_types=[
        pltpu.VMEM((NCHUNK, C), _i32),
        pltpu.VMEM((NCHUNK, C), _i32),
        pltpu.VMEM((NCHUNK, C), _f32),
        [pltpu.VMEM((C, D_HID), _f32) for _ in range(K)],
        [pltpu.VMEM((C, D_HID), _f32) for _ in range(K)],
        pltpu.VMEM_SHARED((NPAD, D_HID), _f32),
        [pltpu.SemaphoreType.DMA for _ in range(K)],
        [pltpu.SemaphoreType.DMA for _ in range(K)],
    ],
  )
  def _s1(hw, row3, col3, val3, zrows, out_ahw,
          rowa, cola, vala, rows, rowso, acc, gsem, ssem):
    c = lax.axis_index("c")
    s = lax.axis_index("s")
    wid = c * NS + s
    rslice = pl.ds(s * RPW, RPW)

    pltpu.sync_copy(row3.at[wid], rowa)
    pltpu.sync_copy(col3.at[wid], cola)
    pltpu.sync_copy(val3.at[wid], vala)
    pltpu.sync_copy(zrows.at[rslice], acc.at[rslice])
    plsc.subcore_barrier()

    def issue_gather(j, b):
        pltpu.async_copy(hw.at[cola.at[j]], rows[b], gsem[b])

    def drain_gather(b):
        pltpu.make_async_copy(hw.at[pl.ds(0, C)], rows[b], gsem[b]).wait()

    def drain_scatter(b):
        pltpu.make_async_copy(hw.at[pl.ds(0, C)], rowso[b], ssem[b]).wait()

    for b in range(K - 1):
        issue_gather(b, b)

    def step(tt, b):
        i = tt * K + b
        j = i + (K - 1)
        bj = (b + K - 1) % K

        if b == 0:
            issue_gather(j, bj)
        else:
            @pl.when(j < NCHUNK)
            def _():
                issue_gather(j, bj)

        drain_gather(b)
        @pl.when(tt >= 1)
        def _():
            drain_scatter(b)
        ivec = jnp.full((16,), i, _i32)
        for e in range(C):
            vs = plsc.load_gather(vala, [ivec, jnp.full((16,), e, _i32)])
            for kk in range(D_HID // 16):
                sl = pl.ds(kk * 16, 16)
                rowso[b][e, sl] = rows[b][e, sl] * vs
        pltpu.async_copy(rowso[b], acc.at[rowa.at[i]], ssem[b], add=True)

    def body(tt, carry):
        for b in range(K):
            step(tt, b)
        return carry

    lax.fori_loop(0, NCHUNK // K, body, 0)
    for b in range(K):
        drain_scatter(b)
    plsc.subcore_barrier()
    pltpu.sync_copy(acc.at[rslice], out_ahw.at[c].at[rslice])

  return _s1


def _t2_body(ahw_ref, b1_ref, wlt_ref, bl_ref, y_ref):
    ahw = ahw_ref[0, :N, :] + ahw_ref[1, :N, :]
    h1 = jnp.maximum(ahw + b1_ref[...], 0.0)
    h2 = jnp.dot(h1, wlt_ref[...], preferred_element_type=_f32) + bl_ref[...]
    h2 = jnp.maximum(h2, 0.0)
    m = jnp.max(h2, axis=1, keepdims=True)
    ex = jnp.exp(h2 - m)
    y = ex / jnp.sum(ex, axis=1, keepdims=True)
    y_ref[...] = y


_t2 = pl.pallas_call(
    _t2_body,
    out_shape=jax.ShapeDtypeStruct((N, G), _f32),
)


@functools.lru_cache(maxsize=None)
def _build_s2():
  mesh = plsc.VectorSubcoreMesh(
      core_axis_name="c", subcore_axis_name="s", num_cores=NC, num_subcores=NS
  )

  @functools.partial(
    pl.kernel,
    out_type=jax.ShapeDtypeStruct((NW, 2, 16), _f32),
    mesh=mesh,
    compiler_params=_sc_params,
    scratch_types=[
        pltpu.VMEM((NCHUNK, C), _i32),
        pltpu.VMEM((NCHUNK, C), _i32),
        pltpu.VMEM((NCHUNK, C), _f32),
        [pltpu.VMEM((C, G), _f32) for _ in range(K)],
        [pltpu.VMEM((C, G), _f32) for _ in range(K)],
        pltpu.VMEM((2, 16), _f32),
        [pltpu.SemaphoreType.DMA for _ in range(K)],
        [pltpu.SemaphoreType.DMA for _ in range(K)],
    ],
  )
  def _s2(ytab, row3, col3, val3, out,
          rowa, cola, vala, ybgr, y1mr, accb, ysem, zsem):
    c = lax.axis_index("c")
    s = lax.axis_index("s")
    wid = c * NS + s

    pltpu.sync_copy(row3.at[wid], rowa)
    pltpu.sync_copy(col3.at[wid], cola)
    pltpu.sync_copy(val3.at[wid], vala)

    def issue_gathers(j, b):
        pltpu.async_copy(ytab.at[rowa.at[j]], ybgr[b], ysem[b])
        pltpu.async_copy(ytab.at[cola.at[j]], y1mr[b], zsem[b])

    def drain_gathers(b):
        pltpu.make_async_copy(ytab.at[pl.ds(0, C)], ybgr[b], ysem[b]).wait()
        pltpu.make_async_copy(ytab.at[pl.ds(0, C)], y1mr[b], zsem[b]).wait()

    for b in range(K - 1):
        issue_gathers(b, b)

    def step(tt, b, carry):
        macc, gacc = carry
        i = tt * K + b
        j = i + (K - 1)
        bj = (b + K - 1) % K
        if b == 0:
            issue_gathers(j, bj)
        else:
            @pl.when(j < NCHUNK)
            def _():
                issue_gathers(j, bj)
        drain_gathers(b)
        for e in range(C):
            vs = plsc.load_gather(
                vala, [jnp.full((16,), i, _i32), jnp.full((16,), e, _i32)]
            )
            t = vs * ybgr[b][e]
            gacc = gacc + t
            macc = macc + t * (1.0 - y1mr[b][e])
        return macc, gacc

    def body(tt, carry):
        for b in range(K):
            carry = step(tt, b, carry)
        return carry

    macc, gacc = lax.fori_loop(
        0, NCHUNK // K, body,
        (jnp.zeros((16,), _f32), jnp.zeros((16,), _f32)),
    )
    accb[0] = macc
    accb[1] = gacc
    pltpu.sync_copy(accb, out.at[wid])

  return _s2


def kernel(H, edge_index, edge_values, W1, b1, Wl, bl):
    row3 = edge_index[0].reshape(NW, NCHUNK, C)
    col3 = edge_index[1].reshape(NW, NCHUNK, C)
    val3 = edge_values.reshape(NW, NCHUNK, C)
    hw = _t1(H, W1)
    zrows = jnp.zeros((NPAD, D_HID), _f32)
    ahwp = _build_s1()(hw, row3, col3, val3, zrows)
    y = _t2(ahwp, b1.reshape(1, D_HID), Wl.T, bl.reshape(1, G))
    parts = _build_s2()(y, row3, col3, val3)
    m = jnp.sum(parts[:, 0, :], axis=0)
    gamma = jnp.sum(parts[:, 1, :], axis=0)
    return jnp.sum(m / gamma).reshape(1)


# speedup vs baseline: 21.4783x; 1.5930x over previous
import functools

import jax
import jax.numpy as jnp
from jax import lax
from jax.experimental import pallas as pl
from jax.experimental.pallas import tpu as pltpu
from jax.experimental.pallas import tpu_sc as plsc

N = 10000
E = 320000
D_IN = 128
D_HID = 64
G = 16

NC = 2
NS = 16
NW = NC * NS
EPT = E // NW
C = 80
NCHUNK = EPT // C
NPAD = 10240
RPW = NPAD // NS
K = 5

_f32 = jnp.float32
_i32 = jnp.int32

_sc_params = pltpu.CompilerParams(
    needs_layout_passes=False, use_tc_tiling_on_sc=False
)


def _t1_body(h_ref, w_ref, o_ref):
    o_ref[...] = jnp.dot(h_ref[...], w_ref[...], preferred_element_type=_f32)


_t1 = pl.pallas_call(
    _t1_body,
    out_shape=jax.ShapeDtypeStruct((N, D_HID), _f32),
)


@functools.lru_cache(maxsize=None)
def _build_s1():
  mesh = plsc.VectorSubcoreMesh(
      core_axis_name="c", subcore_axis_name="s", num_cores=NC, num_subcores=NS
  )

  @functools.partial(
    pl.kernel,
    out_type=jax.ShapeDtypeStruct((NC, NPAD, D_HID), _f32),
    mesh=mesh,
    compiler_params=_sc_params,
    scratch_types=[
        pltpu.VMEM((NCHUNK, C), _i32),
        pltpu.VMEM((NCHUNK, C), _i32),
        pltpu.VMEM((NCHUNK, C), _f32),
        [pltpu.VMEM((C, D_HID), _f32) for _ in range(K)],
        [pltpu.VMEM((C, D_HID), _f32) for _ in range(K)],
        pltpu.VMEM_SHARED((NPAD, D_HID), _f32),
        [pltpu.SemaphoreType.DMA for _ in range(K)],
        [pltpu.SemaphoreType.DMA for _ in range(K)],
    ],
  )
  def _s1(hw, row3, col3, val3, zrows, out_ahw,
          rowa, cola, vala, rows, rowso, acc, gsem, ssem):
    c = lax.axis_index("c")
    s = lax.axis_index("s")
    wid = c * NS + s
    rslice = pl.ds(s * RPW, RPW)

    pltpu.sync_copy(row3.at[wid], rowa)
    pltpu.sync_copy(col3.at[wid], cola)
    pltpu.sync_copy(val3.at[wid], vala)
    pltpu.sync_copy(zrows.at[rslice], acc.at[rslice])
    plsc.subcore_barrier()

    def issue_gather(j, b):
        pltpu.async_copy(hw.at[cola.at[j]], rows[b], gsem[b])

    def drain_gather(b):
        pltpu.make_async_copy(hw.at[pl.ds(0, C)], rows[b], gsem[b]).wait()

    def drain_scatter(b):
        pltpu.make_async_copy(hw.at[pl.ds(0, C)], rowso[b], ssem[b]).wait()

    for b in range(K - 1):
        issue_gather(b, b)

    def step(tt, b):
        i = tt * K + b
        j = i + (K - 1)
        bj = (b + K - 1) % K

        if b == 0:
            issue_gather(j, bj)
        else:
            @pl.when(j < NCHUNK)
            def _():
                issue_gather(j, bj)

        drain_gather(b)
        @pl.when(tt >= 1)
        def _():
            drain_scatter(b)
        for g in range(C // 16):
            vals16 = vala[i, pl.ds(g * 16, 16)]
            for el in range(16):
                e = g * 16 + el
                vs = vals16[el]
                for kk in range(D_HID // 16):
                    sl = pl.ds(kk * 16, 16)
                    rowso[b][e, sl] = rows[b][e, sl] * vs
        pltpu.async_copy(rowso[b], acc.at[rowa.at[i]], ssem[b], add=True)

    def body(tt, carry):
        for b in range(K):
            step(tt, b)
        return carry

    lax.fori_loop(0, NCHUNK // K, body, 0)
    for b in range(K):
        drain_scatter(b)
    plsc.subcore_barrier()
    pltpu.sync_copy(acc.at[rslice], out_ahw.at[c].at[rslice])

  return _s1


def _t2_body(ahw_ref, b1_ref, wlt_ref, bl_ref, y_ref):
    ahw = ahw_ref[0, :N, :] + ahw_ref[1, :N, :]
    h1 = jnp.maximum(ahw + b1_ref[...], 0.0)
    h2 = jnp.dot(h1, wlt_ref[...], preferred_element_type=_f32) + bl_ref[...]
    h2 = jnp.maximum(h2, 0.0)
    m = jnp.max(h2, axis=1, keepdims=True)
    ex = jnp.exp(h2 - m)
    y = ex / jnp.sum(ex, axis=1, keepdims=True)
    y_ref[...] = y


_t2 = pl.pallas_call(
    _t2_body,
    out_shape=jax.ShapeDtypeStruct((N, G), _f32),
)


@functools.lru_cache(maxsize=None)
def _build_s2():
  mesh = plsc.VectorSubcoreMesh(
      core_axis_name="c", subcore_axis_name="s", num_cores=NC, num_subcores=NS
  )

  @functools.partial(
    pl.kernel,
    out_type=jax.ShapeDtypeStruct((NW, 2, 16), _f32),
    mesh=mesh,
    compiler_params=_sc_params,
    scratch_types=[
        pltpu.VMEM((NCHUNK, C), _i32),
        pltpu.VMEM((NCHUNK, C), _i32),
        pltpu.VMEM((NCHUNK, C), _f32),
        [pltpu.VMEM((C, G), _f32) for _ in range(K)],
        [pltpu.VMEM((C, G), _f32) for _ in range(K)],
        pltpu.VMEM((2, 16), _f32),
        [pltpu.SemaphoreType.DMA for _ in range(K)],
        [pltpu.SemaphoreType.DMA for _ in range(K)],
    ],
  )
  def _s2(ytab, row3, col3, val3, out,
          rowa, cola, vala, ybgr, y1mr, accb, ysem, zsem):
    c = lax.axis_index("c")
    s = lax.axis_index("s")
    wid = c * NS + s

    pltpu.sync_copy(row3.at[wid], rowa)
    pltpu.sync_copy(col3.at[wid], cola)
    pltpu.sync_copy(val3.at[wid], vala)

    def issue_gathers(j, b):
        pltpu.async_copy(ytab.at[rowa.at[j]], ybgr[b], ysem[b])
        pltpu.async_copy(ytab.at[cola.at[j]], y1mr[b], zsem[b])

    def drain_gathers(b):
        pltpu.make_async_copy(ytab.at[pl.ds(0, C)], ybgr[b], ysem[b]).wait()
        pltpu.make_async_copy(ytab.at[pl.ds(0, C)], y1mr[b], zsem[b]).wait()

    for b in range(K - 1):
        issue_gathers(b, b)

    def step(tt, b, carry):
        macc, gacc = carry
        i = tt * K + b
        j = i + (K - 1)
        bj = (b + K - 1) % K
        if b == 0:
            issue_gathers(j, bj)
        else:
            @pl.when(j < NCHUNK)
            def _():
                issue_gathers(j, bj)
        drain_gathers(b)
        for g in range(C // 16):
            vals16 = vala[i, pl.ds(g * 16, 16)]
            for el in range(16):
                e = g * 16 + el
                t = vals16[el] * ybgr[b][e]
                gacc = gacc + t
                macc = macc + t * (1.0 - y1mr[b][e])
        return macc, gacc

    def body(tt, carry):
        for b in range(K):
            carry = step(tt, b, carry)
        return carry

    macc, gacc = lax.fori_loop(
        0, NCHUNK // K, body,
        (jnp.zeros((16,), _f32), jnp.zeros((16,), _f32)),
    )
    accb[0] = macc
    accb[1] = gacc
    pltpu.sync_copy(accb, out.at[wid])

  return _s2


def kernel(H, edge_index, edge_values, W1, b1, Wl, bl):
    row3 = edge_index[0].reshape(NW, NCHUNK, C)
    col3 = edge_index[1].reshape(NW, NCHUNK, C)
    val3 = edge_values.reshape(NW, NCHUNK, C)
    hw = _t1(H, W1)
    zrows = jnp.zeros((NPAD, D_HID), _f32)
    ahwp = _build_s1()(hw, row3, col3, val3, zrows)
    y = _t2(ahwp, b1.reshape(1, D_HID), Wl.T, bl.reshape(1, G))
    parts = _build_s2()(y, row3, col3, val3)
    m = jnp.sum(parts[:, 0, :], axis=0)
    gamma = jnp.sum(parts[:, 1, :], axis=0)
    return jnp.sum(m / gamma).reshape(1)
